# baseline (speedup 1.0000x reference)
import jax
import jax.numpy as jnp
from jax.experimental import pallas as pl
from jax.experimental.pallas import tpu as pltpu

N_AGENTS = 4096
OBS_DIM = 256
ACT_DIM = 64
HIDDEN_DIM = 128

BLOCK_I = 512


def _half(adj, h_full, h_blk, w1, b1, w2, b2):
    adjf = adj.astype(jnp.float32)
    deg = jnp.sum(adj, axis=1, keepdims=True).astype(jnp.float32)
    msg_sum = jnp.dot(adjf, h_full, preferred_element_type=jnp.float32)
    msg = jnp.where(deg > 0.0, msg_sum / jnp.maximum(deg, 1.0), 0.0)
    combined = jnp.concatenate([h_blk, msg], axis=-1)
    hidden = jnp.tanh(jnp.dot(combined, w1, preferred_element_type=jnp.float32) + b1)
    return jnp.dot(hidden, w2, preferred_element_type=jnp.float32) + b2


def _fused_kernel(
    obs_ref, adja_ref, adjb_ref, we_ref, be_ref, w1_ref, b1_ref, w2_ref, b2_ref,
    out_ref, h_ref,
):
    i = pl.program_id(0)

    @pl.when(i == 0)
    def _encode():
        h_ref[...] = jnp.tanh(
            jnp.dot(obs_ref[...], we_ref[...], preferred_element_type=jnp.float32)
            + be_ref[...]
        )

    h_full = h_ref[...]
    w1, b1, w2, b2 = w1_ref[...], b1_ref[...], w2_ref[...], b2_ref[...]
    ha = h_ref[pl.ds((2 * i) * BLOCK_I, BLOCK_I), :]
    hb = h_ref[pl.ds((2 * i + 1) * BLOCK_I, BLOCK_I), :]
    out_ref[pl.ds(0, BLOCK_I), :] = _half(adja_ref[...], h_full, ha, w1, b1, w2, b2)
    out_ref[pl.ds(BLOCK_I, BLOCK_I), :] = _half(adjb_ref[...], h_full, hb, w1, b1, w2, b2)


@jax.jit
def kernel(obs_agents, adj, W_enc, b_enc, W1, b1, W2, b2):
    n = N_AGENTS
    b_enc2 = b_enc.reshape(1, HIDDEN_DIM)
    b12 = b1.reshape(1, HIDDEN_DIM)
    b22 = b2.reshape(1, ACT_DIM)

    logits = pl.pallas_call(
        _fused_kernel,
        grid=(n // (2 * BLOCK_I),),
        in_specs=[
            pl.BlockSpec((n, OBS_DIM), lambda i: (0, 0)),
            pl.BlockSpec((BLOCK_I, n), lambda i: (2 * i, 0)),
            pl.BlockSpec((BLOCK_I, n), lambda i: (2 * i + 1, 0)),
            pl.BlockSpec((OBS_DIM, HIDDEN_DIM), lambda i: (0, 0)),
            pl.BlockSpec((1, HIDDEN_DIM), lambda i: (0, 0)),
            pl.BlockSpec((2 * HIDDEN_DIM, HIDDEN_DIM), lambda i: (0, 0)),
            pl.BlockSpec((1, HIDDEN_DIM), lambda i: (0, 0)),
            pl.BlockSpec((HIDDEN_DIM, ACT_DIM), lambda i: (0, 0)),
            pl.BlockSpec((1, ACT_DIM), lambda i: (0, 0)),
        ],
        out_specs=pl.BlockSpec((2 * BLOCK_I, ACT_DIM), lambda i: (i, 0)),
        out_shape=jax.ShapeDtypeStruct((n, ACT_DIM), jnp.float32),
        scratch_shapes=[pltpu.VMEM((n, HIDDEN_DIM), jnp.float32)],
    )(obs_agents, adj, adj, W_enc, b_enc2, W1, b12, W2, b22)

    return logits


# speedup vs baseline: 1.1290x; 1.1290x over previous
import jax
import jax.numpy as jnp
from jax.experimental import pallas as pl
from jax.experimental.pallas import tpu as pltpu

N_AGENTS = 4096
OBS_DIM = 256
ACT_DIM = 64
HIDDEN_DIM = 128

BLOCK_I = 512


def _fused_kernel(
    obs_ref, adj_ref, we_ref, be_ref, w1_ref, b1_ref, w2_ref, b2_ref,
    out_ref, h_ref,
):
    i = pl.program_id(0)

    @pl.when(i == 0)
    def _encode():
        h_ref[...] = jnp.tanh(
            jnp.dot(obs_ref[...], we_ref[...], preferred_element_type=jnp.float32)
            + be_ref[...]
        )

    adjf = adj_ref[...].astype(jnp.float32)
    deg = jnp.sum(adjf, axis=1, keepdims=True)
    msg_sum = jnp.dot(adjf, h_ref[...], preferred_element_type=jnp.float32)
    msg = jnp.where(deg > 0.0, msg_sum / jnp.maximum(deg, 1.0), 0.0)
    h_blk = h_ref[pl.ds(i * BLOCK_I, BLOCK_I), :]
    combined = jnp.concatenate([h_blk, msg], axis=-1)
    hidden = jnp.tanh(
        jnp.dot(combined, w1_ref[...], preferred_element_type=jnp.float32)
        + b1_ref[...]
    )
    out_ref[...] = (
        jnp.dot(hidden, w2_ref[...], preferred_element_type=jnp.float32)
        + b2_ref[...]
    )


@jax.jit
def kernel(obs_agents, adj, W_enc, b_enc, W1, b1, W2, b2):
    n = N_AGENTS
    b_enc2 = b_enc.reshape(1, HIDDEN_DIM)
    b12 = b1.reshape(1, HIDDEN_DIM)
    b22 = b2.reshape(1, ACT_DIM)

    logits = pl.pallas_call(
        _fused_kernel,
        grid=(n // BLOCK_I,),
        in_specs=[
            pl.BlockSpec((n, OBS_DIM), lambda i: (0, 0)),
            pl.BlockSpec((BLOCK_I, n), lambda i: (i, 0)),
            pl.BlockSpec((OBS_DIM, HIDDEN_DIM), lambda i: (0, 0)),
            pl.BlockSpec((1, HIDDEN_DIM), lambda i: (0, 0)),
            pl.BlockSpec((2 * HIDDEN_DIM, HIDDEN_DIM), lambda i: (0, 0)),
            pl.BlockSpec((1, HIDDEN_DIM), lambda i: (0, 0)),
            pl.BlockSpec((HIDDEN_DIM, ACT_DIM), lambda i: (0, 0)),
            pl.BlockSpec((1, ACT_DIM), lambda i: (0, 0)),
        ],
        out_specs=pl.BlockSpec((BLOCK_I, ACT_DIM), lambda i: (i, 0)),
        out_shape=jax.ShapeDtypeStruct((n, ACT_DIM), jnp.float32),
        scratch_shapes=[pltpu.VMEM((n, HIDDEN_DIM), jnp.float32)],
    )(obs_agents, adj, W_enc, b_enc2, W1, b12, W2, b22)

    return logits


# speedup vs baseline: 1.1953x; 1.0588x over previous
import jax
import jax.numpy as jnp
from jax.experimental import pallas as pl
from jax.experimental.pallas import tpu as pltpu

N_AGENTS = 4096
OBS_DIM = 256
ACT_DIM = 64
HIDDEN_DIM = 128

BLOCK_I = 512


def _fused_kernel(
    obs_ref, adj_ref, we_ref, be_ref, w1_ref, b1_ref, w2_ref, b2_ref,
    out_ref, h_ref,
):
    i = pl.program_id(0)

    @pl.when(i == 0)
    def _encode():
        h_ref[...] = jnp.tanh(
            jnp.dot(obs_ref[...], we_ref[...], preferred_element_type=jnp.float32)
            + be_ref[...]
        )

    adj = adj_ref[...]
    adjf = adj.astype(jnp.float32)
    deg = jnp.sum(adj, axis=1, keepdims=True).astype(jnp.float32)
    msg_sum = jnp.dot(adjf, h_ref[...], preferred_element_type=jnp.float32)
    msg = jnp.where(deg > 0.0, msg_sum / jnp.maximum(deg, 1.0), 0.0)
    h_blk = h_ref[pl.ds(i * BLOCK_I, BLOCK_I), :]
    combined = jnp.concatenate([h_blk, msg], axis=-1)
    hidden = jnp.tanh(
        jnp.dot(combined, w1_ref[...], preferred_element_type=jnp.float32)
        + b1_ref[...]
    )
    out_ref[...] = (
        jnp.dot(hidden, w2_ref[...], preferred_element_type=jnp.float32)
        + b2_ref[...]
    )


@jax.jit
def kernel(obs_agents, adj, W_enc, b_enc, W1, b1, W2, b2):
    n = N_AGENTS
    b_enc2 = b_enc.reshape(1, HIDDEN_DIM)
    b12 = b1.reshape(1, HIDDEN_DIM)
    b22 = b2.reshape(1, ACT_DIM)

    logits = pl.pallas_call(
        _fused_kernel,
        grid=(n // BLOCK_I,),
        in_specs=[
            pl.BlockSpec((n, OBS_DIM), lambda i: (0, 0)),
            pl.BlockSpec((BLOCK_I, n), lambda i: (i, 0)),
            pl.BlockSpec((OBS_DIM, HIDDEN_DIM), lambda i: (0, 0)),
            pl.BlockSpec((1, HIDDEN_DIM), lambda i: (0, 0)),
            pl.BlockSpec((2 * HIDDEN_DIM, HIDDEN_DIM), lambda i: (0, 0)),
            pl.BlockSpec((1, HIDDEN_DIM), lambda i: (0, 0)),
            pl.BlockSpec((HIDDEN_DIM, ACT_DIM), lambda i: (0, 0)),
            pl.BlockSpec((1, ACT_DIM), lambda i: (0, 0)),
        ],
        out_specs=pl.BlockSpec((BLOCK_I, ACT_DIM), lambda i: (i, 0)),
        out_shape=jax.ShapeDtypeStruct((n, ACT_DIM), jnp.float32),
        scratch_shapes=[pltpu.VMEM((n, HIDDEN_DIM), jnp.float32)],
    )(obs_agents, adj, W_enc, b_enc2, W1, b12, W2, b22)

    return logits


# speedup vs baseline: 1.1972x; 1.0016x over previous
import jax
import jax.numpy as jnp
from jax.experimental import pallas as pl
from jax.experimental.pallas import tpu as pltpu

N_AGENTS = 4096
OBS_DIM = 256
ACT_DIM = 64
HIDDEN_DIM = 128

BLOCK_I = 512


def _fused_kernel(
    adj_ref, obs_ref, we_ref, be_ref, w1_ref, b1_ref, w2_ref, b2_ref,
    out_ref, h_ref,
):
    i = pl.program_id(0)

    @pl.when(i == 0)
    def _encode():
        h_ref[...] = jnp.tanh(
            jnp.dot(obs_ref[...], we_ref[...], preferred_element_type=jnp.float32)
            + be_ref[...]
        )

    adj = adj_ref[...]
    adjf = adj.astype(jnp.float32)
    deg = jnp.sum(adj, axis=1, keepdims=True).astype(jnp.float32)
    msg_sum = jnp.dot(adjf, h_ref[...], preferred_element_type=jnp.float32)
    msg = jnp.where(deg > 0.0, msg_sum / jnp.maximum(deg, 1.0), 0.0)
    h_blk = h_ref[pl.ds(i * BLOCK_I, BLOCK_I), :]
    combined = jnp.concatenate([h_blk, msg], axis=-1)
    hidden = jnp.tanh(
        jnp.dot(combined, w1_ref[...], preferred_element_type=jnp.float32)
        + b1_ref[...]
    )
    out_ref[...] = (
        jnp.dot(hidden, w2_ref[...], preferred_element_type=jnp.float32)
        + b2_ref[...]
    )


@jax.jit
def kernel(obs_agents, adj, W_enc, b_enc, W1, b1, W2, b2):
    n = N_AGENTS
    b_enc2 = b_enc.reshape(1, HIDDEN_DIM)
    b12 = b1.reshape(1, HIDDEN_DIM)
    b22 = b2.reshape(1, ACT_DIM)

    logits = pl.pallas_call(
        _fused_kernel,
        grid=(n // BLOCK_I,),
        in_specs=[
            pl.BlockSpec((BLOCK_I, n), lambda i: (i, 0)),
            pl.BlockSpec((n, OBS_DIM), lambda i: (0, 0)),
            pl.BlockSpec((OBS_DIM, HIDDEN_DIM), lambda i: (0, 0)),
            pl.BlockSpec((1, HIDDEN_DIM), lambda i: (0, 0)),
            pl.BlockSpec((2 * HIDDEN_DIM, HIDDEN_DIM), lambda i: (0, 0)),
            pl.BlockSpec((1, HIDDEN_DIM), lambda i: (0, 0)),
            pl.BlockSpec((HIDDEN_DIM, ACT_DIM), lambda i: (0, 0)),
            pl.BlockSpec((1, ACT_DIM), lambda i: (0, 0)),
        ],
        out_specs=pl.BlockSpec((BLOCK_I, ACT_DIM), lambda i: (i, 0)),
        out_shape=jax.ShapeDtypeStruct((n, ACT_DIM), jnp.float32),
        scratch_shapes=[pltpu.VMEM((n, HIDDEN_DIM), jnp.float32)],
    )(adj, obs_agents, W_enc, b_enc2, W1, b12, W2, b22)

    return logits
